# baseline (speedup 1.0000x reference)
import functools

import jax
import jax.numpy as jnp
from jax import lax
from jax.experimental import pallas as pl
from jax.experimental.pallas import tpu as pltpu
from jax.experimental.pallas import tpu_sc as plsc

DIM = 64
BATCH = 16384
NUM_E = 1000000

NC = 2
NS = 16
L = 16
NW = NC * NS
B_PER_W = BATCH // NW

SW_ENT = 512
N_SW = NUM_E // SW_ENT
SW_PER_W = 61
LAST_SW = N_SW - 1
TAIL0 = N_SW * SW_ENT
TAIL_SW = N_SW

NLOOK = 3 * BATCH
CHUNK_I = 2048
N_CHUNKS_I = NLOOK // CHUNK_I

LENT_CAP = 1536
LREL_CAP = 1024
WBUF_CAP = 96

DUMP = NLOOK
ROWS = NLOOK + L

C2 = 128


def _phase1(idx_hbm, ent_t, rel_t, tail_e, tail_r, rows_hbm,
            chunk_v, le_e, le_s, lr_e, lr_s, slab3, st3,
            idx_a, idx_b, wb_e, wb_s, tl_e, tl_r,
            sem_i, sem_s, sem_w):
    w = lax.axis_index("s") * NC + lax.axis_index("c")
    lane = lax.iota(jnp.int32, L)
    dump_vec = jnp.full((L,), DUMP, jnp.int32)

    def fill_e(v, _):
        le_e[pl.ds(v * L, L)] = jnp.full((L,), -1, jnp.int32)
        return _
    lax.fori_loop(0, LENT_CAP // L, fill_e, 0)

    def fill_r(v, _):
        lr_e[pl.ds(v * L, L)] = jnp.full((L,), -1, jnp.int32)
        return _
    lax.fori_loop(0, LREL_CAP // L, fill_r, 0)

    cps = [pltpu.async_copy(idx_hbm.at[c], chunk_v.at[c], sem_i)
           for c in range(2)]

    def scan_chunk(c, dst_e, dst_s, n0):
        def step(v, n):
            evec = chunk_v[c % 2, 0, pl.ds(v * L, L)]
            svec = c * CHUNK_I + v * L + lane
            mask = jnp.bitwise_and(jnp.right_shift(evec, 9), NW - 1) == w
            cnt16 = plsc.all_reduce_population_count(mask)
            plsc.store_compressed(dst_e.at[pl.ds(n, L)], evec, mask=mask)
            plsc.store_compressed(dst_s.at[pl.ds(n, L)], svec, mask=mask)
            return n + cnt16[0]
        return lax.fori_loop(0, CHUNK_I // L, step, n0)

    n_e = jnp.int32(0)
    n_r = jnp.int32(0)
    for c in range(N_CHUNKS_I):
        cps[c % 2].wait()
        if 8 <= c < 16:
            n_r = scan_chunk(c, lr_e, lr_s, n_r)
        else:
            n_e = scan_chunk(c, le_e, le_s, n_e)
        if c + 2 < N_CHUNKS_I:
            cps[c % 2] = pltpu.async_copy(idx_hbm.at[c + 2],
                                          chunk_v.at[c % 2], sem_i)

    idx_a[...] = dump_vec
    idx_b[...] = dump_vec
    pltpu.async_copy(st3.at[0], rows_hbm.at[idx_a], sem_w)

    def append_entry(gather_fn, sval, col, cnt):
        par = jnp.bitwise_and(jnp.right_shift(cnt, 4), 1)
        k = jnp.bitwise_and(cnt, L - 1)
        cz = jnp.full((L,), col, jnp.int32)
        for cb in range(DIM // L):
            dvec = cb * L + lane
            st3[par, k, pl.ds(cb * L, L)] = gather_fn(dvec, cz)

        @pl.when(par == 0)
        def _():
            idx_a[...] = jnp.where(lane == k, sval, idx_a[...])

        @pl.when(par == 1)
        def _():
            idx_b[...] = jnp.where(lane == k, sval, idx_b[...])

        cnt = cnt + 1

        @pl.when(jnp.bitwise_and(cnt, L - 1) == 0)
        def _():
            pltpu.make_async_copy(st3.at[0], rows_hbm.at[idx_a],
                                  sem_w).wait()

            @pl.when(par == 0)
            def _():
                pltpu.async_copy(st3.at[0], rows_hbm.at[idx_a], sem_w)
                idx_b[...] = dump_vec

            @pl.when(par == 1)
            def _():
                pltpu.async_copy(st3.at[1], rows_hbm.at[idx_b], sem_w)
                idx_a[...] = dump_vec
        return cnt

    def process_window(sw, gather_fn, colbase, l_ref_e, l_ref_s, lcap,
                       cnt):
        def wfill(v, _):
            wb_e[pl.ds(v * L, L)] = jnp.full((L,), colbase, jnp.int32)
            wb_s[pl.ds(v * L, L)] = dump_vec
            return _
        lax.fori_loop(0, WBUF_CAP // L, wfill, 0)

        def match(v, m):
            evec = l_ref_e[pl.ds(v * L, L)]
            svec = l_ref_s[pl.ds(v * L, L)]
            mask = jnp.right_shift(evec, 9) == sw
            n16 = plsc.all_reduce_population_count(mask)
            plsc.store_compressed(wb_e.at[pl.ds(m, L)], evec, mask=mask)
            plsc.store_compressed(wb_s.at[pl.ds(m, L)], svec, mask=mask)
            return m + n16[0]
        m = lax.fori_loop(0, lcap // L, match, jnp.int32(0))

        def pvec(p, cnt):
            evec = wb_e[pl.ds(p * L, L)]
            svec = wb_s[pl.ds(p * L, L)]
            for i in range(L):
                cnt = append_entry(gather_fn, svec[i], evec[i] - colbase,
                                   cnt)
            return cnt
        return lax.fori_loop(0, jnp.right_shift(m + L - 1, 4), pvec, cnt)

    def slab_gather(par):
        pz = jnp.full((L,), par, jnp.int32)
        return lambda dvec, cz: plsc.load_gather(slab3, [pz, dvec, cz])

    def do_pass(tab, l_ref_e, l_ref_s, lcap, cnt):
        pltpu.async_copy(
            tab.at[:, pl.ds(pl.multiple_of(w * SW_ENT, 128), SW_ENT)],
            slab3.at[0], sem_s)

        def win(j, cnt):
            sw = w + NW * j
            par = jnp.bitwise_and(j, 1)

            @pl.when(j + 1 < SW_PER_W)
            def _():
                off = pl.multiple_of((sw + NW) * SW_ENT, 128)
                pltpu.async_copy(tab.at[:, pl.ds(off, SW_ENT)],
                                 slab3.at[jnp.bitwise_and(j + 1, 1)],
                                 sem_s)
            pltpu.make_async_copy(tab.at[:, pl.ds(0, SW_ENT)],
                                  slab3.at[0], sem_s).wait()
            return process_window(sw, slab_gather(par), sw * SW_ENT,
                                  l_ref_e, l_ref_s, lcap, cnt)
        return lax.fori_loop(0, SW_PER_W, win, cnt)

    cnt = jnp.int32(0)
    cnt = do_pass(ent_t, le_e, le_s, LENT_CAP, cnt)
    cnt = do_pass(rel_t, lr_e, lr_s, LREL_CAP, cnt)

    def flush_tail(cnt):
        pltpu.make_async_copy(st3.at[0], rows_hbm.at[idx_a],
                              sem_w).wait()
        par = jnp.bitwise_and(jnp.right_shift(cnt, 4), 1)

        @pl.when(par == 0)
        def _():
            pltpu.async_copy(st3.at[0], rows_hbm.at[idx_a], sem_w)

        @pl.when(par == 1)
        def _():
            pltpu.async_copy(st3.at[1], rows_hbm.at[idx_b], sem_w)
        pltpu.make_async_copy(st3.at[0], rows_hbm.at[idx_a],
                              sem_w).wait()

    @pl.when(w == 0)
    def _():
        pltpu.sync_copy(ent_t.at[:, pl.ds(LAST_SW * SW_ENT, SW_ENT)],
                        slab3.at[0])
        pltpu.sync_copy(rel_t.at[:, pl.ds(LAST_SW * SW_ENT, SW_ENT)],
                        slab3.at[1])
        c2 = process_window(LAST_SW, slab_gather(jnp.int32(0)),
                            LAST_SW * SW_ENT, le_e, le_s, LENT_CAP, cnt)
        c2 = process_window(LAST_SW, slab_gather(jnp.int32(1)),
                            LAST_SW * SW_ENT, lr_e, lr_s, LREL_CAP, c2)
        flush_tail(c2)

    @pl.when(w == 1)
    def _():
        pltpu.sync_copy(tail_e, tl_e)
        pltpu.sync_copy(tail_r, tl_r)
        ge = lambda dvec, cz: plsc.load_gather(tl_e, [dvec, cz])
        gr = lambda dvec, cz: plsc.load_gather(tl_r, [dvec, cz])
        c2 = process_window(TAIL_SW, ge, TAIL0, le_e, le_s, LENT_CAP,
                            cnt)
        c2 = process_window(TAIL_SW, gr, TAIL0, lr_e, lr_s, LREL_CAP,
                            c2)
        flush_tail(c2)

    @pl.when(w >= 2)
    def _():
        flush_tail(cnt)


def _phase2(rows_hbm, out_hbm, h_a, r_a, t_a, h_b, r_b, t_b, out_v, sem):
    w = lax.axis_index("s") * NC + lax.axis_index("c")
    lane = lax.iota(jnp.int32, L)
    base = w * B_PER_W
    bufs = [(h_a, r_a, t_a), (h_b, r_b, t_b)]

    def fire(c):
        hb, rb, tb = bufs[c % 2]
        off = base + c * C2
        return (pltpu.async_copy(rows_hbm.at[pl.ds(off, C2)], hb, sem),
                pltpu.async_copy(rows_hbm.at[pl.ds(BATCH + off, C2)],
                                 rb, sem),
                pltpu.async_copy(rows_hbm.at[pl.ds(2 * BATCH + off, C2)],
                                 tb, sem))

    def compute(c):
        hb, rb, tb = bufs[c % 2]

        def group(g, carry):
            jl = g * L + lane
            acc = jnp.zeros((L,), jnp.float32)
            for d in range(DIM):
                dz = jnp.full((L,), d, jnp.int32)
                gh = plsc.load_gather(hb, [jl, dz])
                gr = plsc.load_gather(rb, [jl, dz])
                gt = plsc.load_gather(tb, [jl, dz])
                acc = acc + jnp.abs(gh + gr - gt)
            out_v[pl.ds(c * C2 + g * L, L)] = -acc
            return carry
        lax.fori_loop(0, C2 // L, group, 0)

    inflight = fire(0)
    for c in range(B_PER_W // C2):
        nxt = fire(c + 1) if c + 1 < B_PER_W // C2 else ()
        for cp in inflight:
            cp.wait()
        compute(c)
        inflight = nxt
    pltpu.sync_copy(out_v, out_hbm.at[pl.ds(base, B_PER_W)])


@jax.jit
def kernel(triples, entity_emb, relation_emb):
    idx = triples.astype(jnp.int32)
    all_idx = jnp.concatenate([idx[:, 0], idx[:, 1], idx[:, 2]])
    idx3 = all_idx.reshape(N_CHUNKS_I, 1, CHUNK_I)
    ent_t = entity_emb.T
    rel_t = relation_emb.T
    tail_e = ent_t[:, TAIL0:]
    tail_r = rel_t[:, TAIL0:]

    mesh = plsc.VectorSubcoreMesh(core_axis_name="c", subcore_axis_name="s")
    rows = pl.kernel(
        _phase1, mesh=mesh,
        out_type=jax.ShapeDtypeStruct((ROWS, 2 * DIM), jnp.float32),
        scratch_types=[
            pltpu.VMEM((2, 1, CHUNK_I), jnp.int32),
            pltpu.VMEM((LENT_CAP,), jnp.int32),
            pltpu.VMEM((LENT_CAP,), jnp.int32),
            pltpu.VMEM((LREL_CAP,), jnp.int32),
            pltpu.VMEM((LREL_CAP,), jnp.int32),
            pltpu.VMEM((2, DIM, SW_ENT), jnp.float32),
            pltpu.VMEM((2, L, 2 * DIM), jnp.float32),
            pltpu.VMEM((L,), jnp.int32),
            pltpu.VMEM((L,), jnp.int32),
            pltpu.VMEM((WBUF_CAP,), jnp.int32),
            pltpu.VMEM((WBUF_CAP,), jnp.int32),
            pltpu.VMEM((DIM, DIM), jnp.float32),
            pltpu.VMEM((DIM, DIM), jnp.float32),
            pltpu.SemaphoreType.DMA,
            pltpu.SemaphoreType.DMA,
            pltpu.SemaphoreType.DMA,
        ],
        compiler_params=pltpu.CompilerParams(needs_layout_passes=False),
    )(idx3, ent_t, rel_t, tail_e, tail_r)

    return pl.kernel(
        _phase2, mesh=mesh,
        out_type=jax.ShapeDtypeStruct((BATCH,), jnp.float32),
        scratch_types=[
            pltpu.VMEM((C2, 2 * DIM), jnp.float32),
            pltpu.VMEM((C2, 2 * DIM), jnp.float32),
            pltpu.VMEM((C2, 2 * DIM), jnp.float32),
            pltpu.VMEM((C2, 2 * DIM), jnp.float32),
            pltpu.VMEM((C2, 2 * DIM), jnp.float32),
            pltpu.VMEM((C2, 2 * DIM), jnp.float32),
            pltpu.VMEM((B_PER_W,), jnp.float32),
            pltpu.SemaphoreType.DMA,
        ],
        compiler_params=pltpu.CompilerParams(needs_layout_passes=False),
    )(rows)


# speedup vs baseline: 4.2770x; 4.2770x over previous
import functools

import jax
import jax.numpy as jnp
from jax import lax
from jax.experimental import pallas as pl
from jax.experimental.pallas import tpu as pltpu
from jax.experimental.pallas import tpu_sc as plsc

DIM = 64
BATCH = 16384
NUM_E = 1000000

NC = 2
NS = 16
L = 16
NW = NC * NS
B_PER_W = BATCH // NW

SW_ENT = 512
N_SW = NUM_E // SW_ENT
SW_PER_W = 61
LAST_SW = N_SW - 1
TAIL0 = N_SW * SW_ENT
TAIL_SW = N_SW

NLOOK = 3 * BATCH
CHUNK_I = 2048
N_CHUNKS_I = NLOOK // CHUNK_I

LENT_CAP = 1536
LREL_CAP = 1024
WBUF_CAP = 96
NBKT = 8
BE_CAP = 320
BR_CAP = 192

DUMP = NLOOK
R_CAP = 3072
ROWS2 = NW * R_CAP
POSN = NLOOK + L

C2 = 128


def _phase1(idx_hbm, ent_t, rel_t, tail_e, tail_r, rows_hbm, slot_hbm,
            chunk_v, le_e, le_s, lr_e, lr_s, slab3, st3,
            sl_v, wb_e, wb_s, tl_e, tl_r,
            be_e, be_s, br_e, br_s, bcnt_s,
            sem_i, sem_s, sem_w):
    w = lax.axis_index("s") * NC + lax.axis_index("c")
    lane = lax.iota(jnp.int32, L)
    dump_vec = jnp.full((L,), DUMP, jnp.int32)

    def fill_e(v, _):
        le_e[pl.ds(v * L, L)] = jnp.full((L,), -1, jnp.int32)
        return _
    lax.fori_loop(0, LENT_CAP // L, fill_e, 0)

    def fill_r(v, _):
        lr_e[pl.ds(v * L, L)] = jnp.full((L,), -1, jnp.int32)
        return _
    lax.fori_loop(0, LREL_CAP // L, fill_r, 0)

    def fill_sl(v, _):
        sl_v[v, :] = jnp.full((L,), DUMP, jnp.int32)
        return _
    lax.fori_loop(0, R_CAP // L, fill_sl, 0)

    cps = [pltpu.async_copy(idx_hbm.at[c], chunk_v.at[c], sem_i)
           for c in range(2)]

    def scan_chunk(c, dst_e, dst_s, n0):
        def step(v, n):
            evec = chunk_v[c % 2, 0, pl.ds(v * L, L)]
            svec = c * CHUNK_I + v * L + lane
            mask = jnp.bitwise_and(jnp.right_shift(evec, 9), NW - 1) == w
            cnt16 = plsc.all_reduce_population_count(mask)
            plsc.store_compressed(dst_e.at[pl.ds(n, L)], evec, mask=mask)
            plsc.store_compressed(dst_s.at[pl.ds(n, L)], svec, mask=mask)
            return n + cnt16[0]
        return lax.fori_loop(0, CHUNK_I // L, step, n0)

    n_e = jnp.int32(0)
    n_r = jnp.int32(0)
    for c in range(N_CHUNKS_I):
        cps[c % 2].wait()
        if 8 <= c < 16:
            n_r = scan_chunk(c, lr_e, lr_s, n_r)
        else:
            n_e = scan_chunk(c, le_e, le_s, n_e)
        if c + 2 < N_CHUNKS_I:
            cps[c % 2] = pltpu.async_copy(idx_hbm.at[c + 2],
                                          chunk_v.at[c % 2], sem_i)

    def bfill(dst, cap):
        def f(v, _):
            dst[pl.ds(v * L, L)] = jnp.full((L,), -1, jnp.int32)
            return _
        lax.fori_loop(0, cap // L, f, 0)
    bfill(be_e, NBKT * BE_CAP)
    bfill(br_e, NBKT * BR_CAP)

    def bucketize(src_e, src_s, n, dst_e, dst_s, cap, smem_base):
        nv = jnp.right_shift(n + L - 1, 4)
        for b in range(NBKT):
            def bstep(v, m):
                evec = src_e[pl.ds(v * L, L)]
                svec = src_s[pl.ds(v * L, L)]
                mask = jnp.right_shift(evec, 17) == b
                n16 = plsc.all_reduce_population_count(mask)
                plsc.store_compressed(dst_e.at[pl.ds(b * cap + m, L)],
                                      evec, mask=mask)
                plsc.store_compressed(dst_s.at[pl.ds(b * cap + m, L)],
                                      svec, mask=mask)
                return m + n16[0]
            bn = lax.fori_loop(0, nv, bstep, jnp.int32(0))
            bcnt_s[smem_base + b] = bn

    bucketize(le_e, le_s, n_e, be_e, be_s, BE_CAP, 0)
    bucketize(lr_e, lr_s, n_r, br_e, br_s, BR_CAP, NBKT)

    rbase = w * R_CAP
    pltpu.async_copy(st3.at[0], rows_hbm.at[pl.ds(
        pl.multiple_of(rbase, 8), L)], sem_w)

    def append_entry(gather_fn, col, cnt):
        par = jnp.bitwise_and(jnp.right_shift(cnt, 4), 1)
        k = jnp.bitwise_and(cnt, L - 1)
        cz = jnp.full((L,), col, jnp.int32)
        for cb in range(DIM // L):
            dvec = cb * L + lane
            st3[par, k, pl.ds(cb * L, L)] = gather_fn(dvec, cz)
        cnt = cnt + 1

        @pl.when(jnp.bitwise_and(cnt, L - 1) == 0)
        def _():
            pltpu.make_async_copy(st3.at[0], rows_hbm.at[pl.ds(
                pl.multiple_of(rbase, 8), L)], sem_w).wait()
            dst = pl.ds(pl.multiple_of(rbase + cnt - L, 8), L)

            @pl.when(par == 0)
            def _():
                pltpu.async_copy(st3.at[0], rows_hbm.at[dst], sem_w)

            @pl.when(par == 1)
            def _():
                pltpu.async_copy(st3.at[1], rows_hbm.at[dst], sem_w)
        return cnt

    def process_window(sw, gather_fn, colbase, bkt_e, bkt_s, bcap,
                       smem_base, j, cnt):
        def wfill(v, _):
            wb_e[pl.ds(v * L, L)] = jnp.full((L,), colbase, jnp.int32)
            wb_s[pl.ds(v * L, L)] = dump_vec
            return _
        lax.fori_loop(0, WBUF_CAP // L, wfill, 0)

        b = jnp.right_shift(j, 3)
        boff = b * bcap
        bn = bcnt_s[smem_base + b]

        def match(v, m):
            evec = bkt_e[pl.ds(boff + v * L, L)]
            svec = bkt_s[pl.ds(boff + v * L, L)]
            mask = jnp.right_shift(evec, 9) == sw
            n16 = plsc.all_reduce_population_count(mask)
            plsc.store_compressed(wb_e.at[pl.ds(m, L)], evec, mask=mask)
            plsc.store_compressed(wb_s.at[pl.ds(m, L)], svec, mask=mask)
            return m + n16[0]
        m = lax.fori_loop(0, jnp.right_shift(bn + L - 1, 4), match,
                          jnp.int32(0))

        def pvec(p, cnt):
            evec = wb_e[pl.ds(p * L, L)]
            sl_v[jnp.right_shift(cnt, 4), :] = wb_s[pl.ds(p * L, L)]
            for i in range(L):
                cnt = append_entry(gather_fn, evec[i] - colbase, cnt)
            return cnt
        return lax.fori_loop(0, jnp.right_shift(m + L - 1, 4), pvec, cnt)

    def slab_gather(par):
        pz = jnp.full((L,), par, jnp.int32)
        return lambda dvec, cz: plsc.load_gather(slab3, [pz, dvec, cz])

    def do_pass(tab, bkt_e, bkt_s, bcap, smem_base, cnt):
        pltpu.async_copy(
            tab.at[:, pl.ds(pl.multiple_of(w * SW_ENT, 128), SW_ENT)],
            slab3.at[0], sem_s)

        def win(j, cnt):
            sw = w + NW * j
            par = jnp.bitwise_and(j, 1)

            @pl.when(j + 1 < SW_PER_W)
            def _():
                off = pl.multiple_of((sw + NW) * SW_ENT, 128)
                pltpu.async_copy(tab.at[:, pl.ds(off, SW_ENT)],
                                 slab3.at[jnp.bitwise_and(j + 1, 1)],
                                 sem_s)
            pltpu.make_async_copy(tab.at[:, pl.ds(0, SW_ENT)],
                                  slab3.at[0], sem_s).wait()
            return process_window(sw, slab_gather(par), sw * SW_ENT,
                                  bkt_e, bkt_s, bcap, smem_base, j, cnt)
        return lax.fori_loop(0, SW_PER_W, win, cnt)

    cnt = jnp.int32(0)
    cnt = do_pass(ent_t, be_e, be_s, BE_CAP, 0, cnt)
    cnt = do_pass(rel_t, br_e, br_s, BR_CAP, NBKT, cnt)

    def flush_tail(cnt):
        pltpu.make_async_copy(st3.at[0], rows_hbm.at[pl.ds(
            pl.multiple_of(rbase, 8), L)], sem_w).wait()
        pltpu.sync_copy(sl_v, slot_hbm.at[pl.ds(
            pl.multiple_of(w * (R_CAP // L), 8), R_CAP // L)])

    @pl.when(w == 0)
    def _():
        pltpu.sync_copy(ent_t.at[:, pl.ds(LAST_SW * SW_ENT, SW_ENT)],
                        slab3.at[0])
        pltpu.sync_copy(rel_t.at[:, pl.ds(LAST_SW * SW_ENT, SW_ENT)],
                        slab3.at[1])
        c2 = process_window(LAST_SW, slab_gather(jnp.int32(0)),
                            LAST_SW * SW_ENT, be_e, be_s, BE_CAP, 0,
                            jnp.int32(LAST_SW >> 5), cnt)
        c2 = process_window(LAST_SW, slab_gather(jnp.int32(1)),
                            LAST_SW * SW_ENT, br_e, br_s, BR_CAP, NBKT,
                            jnp.int32(LAST_SW >> 5), c2)
        flush_tail(c2)

    @pl.when(w == 1)
    def _():
        pltpu.sync_copy(tail_e, tl_e)
        pltpu.sync_copy(tail_r, tl_r)
        ge = lambda dvec, cz: plsc.load_gather(tl_e, [dvec, cz])
        gr = lambda dvec, cz: plsc.load_gather(tl_r, [dvec, cz])
        c2 = process_window(TAIL_SW, ge, TAIL0, be_e, be_s, BE_CAP, 0,
                            jnp.int32(TAIL_SW >> 5), cnt)
        c2 = process_window(TAIL_SW, gr, TAIL0, br_e, br_s, BR_CAP, NBKT,
                            jnp.int32(TAIL_SW >> 5), c2)
        flush_tail(c2)

    @pl.when(w >= 2)
    def _():
        flush_tail(cnt)


def _phase2(rows_hbm, slot_hbm, out_hbm, h_a, r_a, t_a,
            sc_v, pos_l, gi, out_v, sem_i, sem):
    w = lax.axis_index("s") * NC + lax.axis_index("c")
    lane = lax.iota(jnp.int32, L)
    base = w * B_PER_W

    NSC = ROWS2 // L // 128
    for c in range(2):
        pltpu.async_copy(slot_hbm.at[pl.ds(c * 128, 128)], sc_v.at[c],
                         sem_i)

    def cbody(c, carry):
        par = jnp.bitwise_and(c, 1)
        pltpu.make_async_copy(slot_hbm.at[pl.ds(0, 128)], sc_v.at[0],
                              sem_i).wait()

        def inv(v, __):
            svec = sc_v[par, v, :]
            gpos = (c * 128 + v) * L + lane
            comp = jnp.right_shift(svec, 14)
            local = jnp.bitwise_and(svec, BATCH - 1) - base
            ok = jnp.logical_and(svec < NLOOK,
                                 jnp.logical_and(local >= 0,
                                                 local < B_PER_W))
            dst = comp * B_PER_W + jnp.clip(local, 0, B_PER_W - 1)
            plsc.store_scatter(pos_l, [dst], gpos, mask=ok)
            return __
        lax.fori_loop(0, 128, inv, 0)

        @pl.when(c + 2 < NSC)
        def _():
            off = pl.multiple_of((c + 2) * 128, 128)
            pltpu.async_copy(slot_hbm.at[pl.ds(off, 128)], sc_v.at[par],
                             sem_i)
        return carry
    lax.fori_loop(0, NSC, cbody, 0)

    def gbody(c, carry):
        for comp, dst in ((0, h_a), (1, r_a), (2, t_a)):
            def gfill(v, __):
                gi[pl.ds(v * L, L)] = pos_l[
                    pl.ds(comp * B_PER_W + c * C2 + v * L, L)]
                return __
            lax.fori_loop(0, C2 // L, gfill, 0)
            pltpu.async_copy(rows_hbm.at[gi], dst, sem).wait()

        def group(g, __):
            jl = g * L + lane
            acc = jnp.zeros((L,), jnp.float32)
            for d in range(DIM):
                dz = jnp.full((L,), d, jnp.int32)
                gh = plsc.load_gather(h_a, [jl, dz])
                gr = plsc.load_gather(r_a, [jl, dz])
                gt = plsc.load_gather(t_a, [jl, dz])
                acc = acc + jnp.abs(gh + gr - gt)
            out_v[pl.ds(c * C2 + g * L, L)] = -acc
            return __
        lax.fori_loop(0, C2 // L, group, 0)
        return carry
    lax.fori_loop(0, B_PER_W // C2, gbody, 0)
    pltpu.sync_copy(out_v, out_hbm.at[pl.ds(base, B_PER_W)])


@jax.jit
def kernel(triples, entity_emb, relation_emb):
    idx = triples.astype(jnp.int32)
    all_idx = jnp.concatenate([idx[:, 0], idx[:, 1], idx[:, 2]])
    idx3 = all_idx.reshape(N_CHUNKS_I, 1, CHUNK_I)
    ent_t = entity_emb.T
    rel_t = relation_emb.T
    tail_e = ent_t[:, TAIL0:]
    tail_r = rel_t[:, TAIL0:]

    mesh = plsc.VectorSubcoreMesh(core_axis_name="c", subcore_axis_name="s")
    rows, slots = pl.kernel(
        _phase1, mesh=mesh,
        out_type=(jax.ShapeDtypeStruct((ROWS2, 2 * DIM), jnp.float32),
                  jax.ShapeDtypeStruct((ROWS2 // L, L), jnp.int32)),
        scratch_types=[
            pltpu.VMEM((2, 1, CHUNK_I), jnp.int32),
            pltpu.VMEM((LENT_CAP,), jnp.int32),
            pltpu.VMEM((LENT_CAP,), jnp.int32),
            pltpu.VMEM((LREL_CAP,), jnp.int32),
            pltpu.VMEM((LREL_CAP,), jnp.int32),
            pltpu.VMEM((2, DIM, SW_ENT), jnp.float32),
            pltpu.VMEM((2, L, 2 * DIM), jnp.float32),
            pltpu.VMEM((R_CAP // L, L), jnp.int32),
            pltpu.VMEM((WBUF_CAP,), jnp.int32),
            pltpu.VMEM((WBUF_CAP,), jnp.int32),
            pltpu.VMEM((DIM, DIM), jnp.float32),
            pltpu.VMEM((DIM, DIM), jnp.float32),
            pltpu.VMEM((NBKT * BE_CAP,), jnp.int32),
            pltpu.VMEM((NBKT * BE_CAP,), jnp.int32),
            pltpu.VMEM((NBKT * BR_CAP,), jnp.int32),
            pltpu.VMEM((NBKT * BR_CAP,), jnp.int32),
            pltpu.SMEM((2 * NBKT,), jnp.int32),
            pltpu.SemaphoreType.DMA,
            pltpu.SemaphoreType.DMA,
            pltpu.SemaphoreType.DMA,
        ],
        compiler_params=pltpu.CompilerParams(needs_layout_passes=False),
    )(idx3, ent_t, rel_t, tail_e, tail_r)

    return pl.kernel(
        _phase2, mesh=mesh,
        out_type=jax.ShapeDtypeStruct((BATCH,), jnp.float32),
        scratch_types=[
            pltpu.VMEM((C2, 2 * DIM), jnp.float32),
            pltpu.VMEM((C2, 2 * DIM), jnp.float32),
            pltpu.VMEM((C2, 2 * DIM), jnp.float32),
            pltpu.VMEM((2, 128, L), jnp.int32),
            pltpu.VMEM((3 * B_PER_W,), jnp.int32),
            pltpu.VMEM((C2,), jnp.int32),
            pltpu.VMEM((B_PER_W,), jnp.float32),
            pltpu.SemaphoreType.DMA,
            pltpu.SemaphoreType.DMA,
        ],
        compiler_params=pltpu.CompilerParams(needs_layout_passes=False),
    )(rows, slots)


# speedup vs baseline: 4.3387x; 1.0144x over previous
import functools

import jax
import jax.numpy as jnp
from jax import lax
from jax.experimental import pallas as pl
from jax.experimental.pallas import tpu as pltpu
from jax.experimental.pallas import tpu_sc as plsc

DIM = 64
BATCH = 16384
NUM_E = 1000000

NC = 2
NS = 16
L = 16
NW = NC * NS
B_PER_W = BATCH // NW

SW_ENT = 512
N_SW = NUM_E // SW_ENT
SW_PER_W = 61
LAST_SW = N_SW - 1
TAIL0 = N_SW * SW_ENT
TAIL_SW = N_SW

NLOOK = 3 * BATCH
CHUNK_I = 2048
N_CHUNKS_I = NLOOK // CHUNK_I

LENT_CAP = 1536
LREL_CAP = 1024
WBUF_CAP = 96
NBKT = 8
BE_CAP = 320
BR_CAP = 192

DUMP = NLOOK
R_CAP = 3072
ROWS2 = NW * R_CAP
POSN = NLOOK + L

C2 = 128


def _phase1(idx_hbm, ent_t, rel_t, tail_e, tail_r, rows_hbm, slot_hbm,
            chunk_v, le_e, le_s, lr_e, lr_s, slab3, st3,
            sl_v, wb_e, wb_s, tl_e, tl_r,
            be_e, be_s, br_e, br_s, bcnt_s,
            sem_i, sem_s, sem_w):
    w = lax.axis_index("s") * NC + lax.axis_index("c")
    lane = lax.iota(jnp.int32, L)
    dump_vec = jnp.full((L,), DUMP, jnp.int32)

    def fill_e(v, _):
        le_e[pl.ds(v * L, L)] = jnp.full((L,), -1, jnp.int32)
        return _
    lax.fori_loop(0, LENT_CAP // L, fill_e, 0)

    def fill_r(v, _):
        lr_e[pl.ds(v * L, L)] = jnp.full((L,), -1, jnp.int32)
        return _
    lax.fori_loop(0, LREL_CAP // L, fill_r, 0)

    def fill_sl(v, _):
        sl_v[v, :] = jnp.full((L,), DUMP, jnp.int32)
        return _
    lax.fori_loop(0, R_CAP // L, fill_sl, 0)

    cps = [pltpu.async_copy(idx_hbm.at[c], chunk_v.at[c], sem_i)
           for c in range(2)]

    def scan_chunk(c, dst_e, dst_s, n0):
        def step(v, n):
            evec = chunk_v[c % 2, 0, pl.ds(v * L, L)]
            svec = c * CHUNK_I + v * L + lane
            mask = jnp.bitwise_and(jnp.right_shift(evec, 9), NW - 1) == w
            cnt16 = plsc.all_reduce_population_count(mask)
            plsc.store_compressed(dst_e.at[pl.ds(n, L)], evec, mask=mask)
            plsc.store_compressed(dst_s.at[pl.ds(n, L)], svec, mask=mask)
            return n + cnt16[0]
        return lax.fori_loop(0, CHUNK_I // L, step, n0)

    n_e = jnp.int32(0)
    n_r = jnp.int32(0)
    for c in range(N_CHUNKS_I):
        cps[c % 2].wait()
        if 8 <= c < 16:
            n_r = scan_chunk(c, lr_e, lr_s, n_r)
        else:
            n_e = scan_chunk(c, le_e, le_s, n_e)
        if c + 2 < N_CHUNKS_I:
            cps[c % 2] = pltpu.async_copy(idx_hbm.at[c + 2],
                                          chunk_v.at[c % 2], sem_i)

    def bfill(dst, cap):
        def f(v, _):
            dst[pl.ds(v * L, L)] = jnp.full((L,), -1, jnp.int32)
            return _
        lax.fori_loop(0, cap // L, f, 0)
    bfill(be_e, NBKT * BE_CAP)
    bfill(br_e, NBKT * BR_CAP)

    def bucketize(src_e, src_s, n, dst_e, dst_s, cap, smem_base):
        nv = jnp.right_shift(n + L - 1, 4)
        for b in range(NBKT):
            def bstep(v, m):
                evec = src_e[pl.ds(v * L, L)]
                svec = src_s[pl.ds(v * L, L)]
                mask = jnp.right_shift(evec, 17) == b
                n16 = plsc.all_reduce_population_count(mask)
                plsc.store_compressed(dst_e.at[pl.ds(b * cap + m, L)],
                                      evec, mask=mask)
                plsc.store_compressed(dst_s.at[pl.ds(b * cap + m, L)],
                                      svec, mask=mask)
                return m + n16[0]
            bn = lax.fori_loop(0, nv, bstep, jnp.int32(0))
            bcnt_s[smem_base + b] = bn

    bucketize(le_e, le_s, n_e, be_e, be_s, BE_CAP, 0)
    bucketize(lr_e, lr_s, n_r, br_e, br_s, BR_CAP, NBKT)

    rbase = w * R_CAP
    pltpu.async_copy(st3.at[0], rows_hbm.at[pl.ds(
        pl.multiple_of(rbase, 8), L)], sem_w)

    def append_entry(gather_fn, col, cnt):
        par = jnp.bitwise_and(jnp.right_shift(cnt, 4), 1)
        k = jnp.bitwise_and(cnt, L - 1)
        cz = jnp.full((L,), col, jnp.int32)
        for cb in range(DIM // L):
            dvec = cb * L + lane
            st3[par, k, pl.ds(cb * L, L)] = gather_fn(dvec, cz)
        cnt = cnt + 1

        @pl.when(jnp.bitwise_and(cnt, L - 1) == 0)
        def _():
            pltpu.make_async_copy(st3.at[0], rows_hbm.at[pl.ds(
                pl.multiple_of(rbase, 8), L)], sem_w).wait()
            dst = pl.ds(pl.multiple_of(rbase + cnt - L, 8), L)

            @pl.when(par == 0)
            def _():
                pltpu.async_copy(st3.at[0], rows_hbm.at[dst], sem_w)

            @pl.when(par == 1)
            def _():
                pltpu.async_copy(st3.at[1], rows_hbm.at[dst], sem_w)
        return cnt

    def process_window(sw, gather_fn, colbase, bkt_e, bkt_s, bcap,
                       smem_base, j, cnt):
        def wfill(v, _):
            wb_e[pl.ds(v * L, L)] = jnp.full((L,), colbase, jnp.int32)
            wb_s[pl.ds(v * L, L)] = dump_vec
            return _
        lax.fori_loop(0, WBUF_CAP // L, wfill, 0)

        b = jnp.right_shift(j, 3)
        boff = b * bcap
        bn = bcnt_s[smem_base + b]

        def match(v, m):
            evec = bkt_e[pl.ds(boff + v * L, L)]
            svec = bkt_s[pl.ds(boff + v * L, L)]
            mask = jnp.right_shift(evec, 9) == sw
            n16 = plsc.all_reduce_population_count(mask)
            plsc.store_compressed(wb_e.at[pl.ds(m, L)], evec, mask=mask)
            plsc.store_compressed(wb_s.at[pl.ds(m, L)], svec, mask=mask)
            return m + n16[0]
        m = lax.fori_loop(0, jnp.right_shift(bn + L - 1, 4), match,
                          jnp.int32(0))

        def pvec(p, cnt):
            evec = wb_e[pl.ds(p * L, L)]
            sl_v[jnp.right_shift(cnt, 4), :] = wb_s[pl.ds(p * L, L)]
            for i in range(L):
                cnt = append_entry(gather_fn, evec[i] - colbase, cnt)
            return cnt
        return lax.fori_loop(0, jnp.right_shift(m + L - 1, 4), pvec, cnt)

    def slab_gather(par):
        pz = jnp.full((L,), par, jnp.int32)
        return lambda dvec, cz: plsc.load_gather(slab3, [pz, dvec, cz])

    def do_pass(tab, bkt_e, bkt_s, bcap, smem_base, cnt):
        pltpu.async_copy(
            tab.at[:, pl.ds(pl.multiple_of(w * SW_ENT, 128), SW_ENT)],
            slab3.at[0], sem_s)

        def win(j, cnt):
            sw = w + NW * j
            par = jnp.bitwise_and(j, 1)

            @pl.when(j + 1 < SW_PER_W)
            def _():
                off = pl.multiple_of((sw + NW) * SW_ENT, 128)
                pltpu.async_copy(tab.at[:, pl.ds(off, SW_ENT)],
                                 slab3.at[jnp.bitwise_and(j + 1, 1)],
                                 sem_s)
            pltpu.make_async_copy(tab.at[:, pl.ds(0, SW_ENT)],
                                  slab3.at[0], sem_s).wait()
            return process_window(sw, slab_gather(par), sw * SW_ENT,
                                  bkt_e, bkt_s, bcap, smem_base, j, cnt)
        return lax.fori_loop(0, SW_PER_W, win, cnt)

    cnt = jnp.int32(0)
    cnt = do_pass(ent_t, be_e, be_s, BE_CAP, 0, cnt)
    cnt = do_pass(rel_t, br_e, br_s, BR_CAP, NBKT, cnt)

    def flush_tail(cnt):
        pltpu.make_async_copy(st3.at[0], rows_hbm.at[pl.ds(
            pl.multiple_of(rbase, 8), L)], sem_w).wait()
        pltpu.sync_copy(sl_v, slot_hbm.at[pl.ds(
            pl.multiple_of(w * (R_CAP // L), 8), R_CAP // L)])

    @pl.when(w == 0)
    def _():
        pltpu.sync_copy(ent_t.at[:, pl.ds(LAST_SW * SW_ENT, SW_ENT)],
                        slab3.at[0])
        pltpu.sync_copy(rel_t.at[:, pl.ds(LAST_SW * SW_ENT, SW_ENT)],
                        slab3.at[1])
        c2 = process_window(LAST_SW, slab_gather(jnp.int32(0)),
                            LAST_SW * SW_ENT, be_e, be_s, BE_CAP, 0,
                            jnp.int32(LAST_SW >> 5), cnt)
        c2 = process_window(LAST_SW, slab_gather(jnp.int32(1)),
                            LAST_SW * SW_ENT, br_e, br_s, BR_CAP, NBKT,
                            jnp.int32(LAST_SW >> 5), c2)
        flush_tail(c2)

    @pl.when(w == 1)
    def _():
        pltpu.sync_copy(tail_e, tl_e)
        pltpu.sync_copy(tail_r, tl_r)
        ge = lambda dvec, cz: plsc.load_gather(tl_e, [dvec, cz])
        gr = lambda dvec, cz: plsc.load_gather(tl_r, [dvec, cz])
        c2 = process_window(TAIL_SW, ge, TAIL0, be_e, be_s, BE_CAP, 0,
                            jnp.int32(TAIL_SW >> 5), cnt)
        c2 = process_window(TAIL_SW, gr, TAIL0, br_e, br_s, BR_CAP, NBKT,
                            jnp.int32(TAIL_SW >> 5), c2)
        flush_tail(c2)

    @pl.when(w >= 2)
    def _():
        flush_tail(cnt)


def _phase2(rows_hbm, slot_hbm, out_hbm, h_a, r_a, t_a,
            sc_v, pos_l, gi0, gi1, gi2, out_v, sem_i, sem):
    w = lax.axis_index("s") * NC + lax.axis_index("c")
    lane = lax.iota(jnp.int32, L)
    base = w * B_PER_W

    NSC = ROWS2 // L // 128
    for c in range(2):
        pltpu.async_copy(slot_hbm.at[pl.ds(c * 128, 128)], sc_v.at[c],
                         sem_i)

    def cbody(c, carry):
        par = jnp.bitwise_and(c, 1)
        pltpu.make_async_copy(slot_hbm.at[pl.ds(0, 128)], sc_v.at[0],
                              sem_i).wait()

        def inv(v, __):
            svec = sc_v[par, v, :]
            gpos = (c * 128 + v) * L + lane
            comp = jnp.right_shift(svec, 14)
            local = jnp.bitwise_and(svec, BATCH - 1) - base
            ok = jnp.logical_and(svec < NLOOK,
                                 jnp.logical_and(local >= 0,
                                                 local < B_PER_W))
            dst = comp * B_PER_W + jnp.clip(local, 0, B_PER_W - 1)
            plsc.store_scatter(pos_l, [dst], gpos, mask=ok)
            return __
        lax.fori_loop(0, 128, inv, 0)

        @pl.when(c + 2 < NSC)
        def _():
            off = pl.multiple_of((c + 2) * 128, 128)
            pltpu.async_copy(slot_hbm.at[pl.ds(off, 128)], sc_v.at[par],
                             sem_i)
        return carry
    lax.fori_loop(0, NSC, cbody, 0)

    def gbody(c, carry):
        cps = []
        for comp, dst, gi in ((0, h_a, gi0), (1, r_a, gi1),
                              (2, t_a, gi2)):
            def gfill(v, __, gi=gi, comp=comp):
                gi[pl.ds(v * L, L)] = pos_l[
                    pl.ds(comp * B_PER_W + c * C2 + v * L, L)]
                return __
            lax.fori_loop(0, C2 // L, gfill, 0)
            cps.append(pltpu.async_copy(rows_hbm.at[gi], dst, sem))
        for cp in cps:
            cp.wait()

        def group(g, __):
            jl = g * L + lane
            acc = jnp.zeros((L,), jnp.float32)
            for d in range(DIM):
                dz = jnp.full((L,), d, jnp.int32)
                gh = plsc.load_gather(h_a, [jl, dz])
                gr = plsc.load_gather(r_a, [jl, dz])
                gt = plsc.load_gather(t_a, [jl, dz])
                acc = acc + jnp.abs(gh + gr - gt)
            out_v[pl.ds(c * C2 + g * L, L)] = -acc
            return __
        lax.fori_loop(0, C2 // L, group, 0)
        return carry
    lax.fori_loop(0, B_PER_W // C2, gbody, 0)
    pltpu.sync_copy(out_v, out_hbm.at[pl.ds(base, B_PER_W)])


@jax.jit
def kernel(triples, entity_emb, relation_emb):
    idx = triples.astype(jnp.int32)
    all_idx = jnp.concatenate([idx[:, 0], idx[:, 1], idx[:, 2]])
    idx3 = all_idx.reshape(N_CHUNKS_I, 1, CHUNK_I)
    ent_t = entity_emb.T
    rel_t = relation_emb.T
    tail_e = ent_t[:, TAIL0:]
    tail_r = rel_t[:, TAIL0:]

    mesh = plsc.VectorSubcoreMesh(core_axis_name="c", subcore_axis_name="s")
    rows, slots = pl.kernel(
        _phase1, mesh=mesh,
        out_type=(jax.ShapeDtypeStruct((ROWS2, 2 * DIM), jnp.float32),
                  jax.ShapeDtypeStruct((ROWS2 // L, L), jnp.int32)),
        scratch_types=[
            pltpu.VMEM((2, 1, CHUNK_I), jnp.int32),
            pltpu.VMEM((LENT_CAP,), jnp.int32),
            pltpu.VMEM((LENT_CAP,), jnp.int32),
            pltpu.VMEM((LREL_CAP,), jnp.int32),
            pltpu.VMEM((LREL_CAP,), jnp.int32),
            pltpu.VMEM((2, DIM, SW_ENT), jnp.float32),
            pltpu.VMEM((2, L, 2 * DIM), jnp.float32),
            pltpu.VMEM((R_CAP // L, L), jnp.int32),
            pltpu.VMEM((WBUF_CAP,), jnp.int32),
            pltpu.VMEM((WBUF_CAP,), jnp.int32),
            pltpu.VMEM((DIM, DIM), jnp.float32),
            pltpu.VMEM((DIM, DIM), jnp.float32),
            pltpu.VMEM((NBKT * BE_CAP,), jnp.int32),
            pltpu.VMEM((NBKT * BE_CAP,), jnp.int32),
            pltpu.VMEM((NBKT * BR_CAP,), jnp.int32),
            pltpu.VMEM((NBKT * BR_CAP,), jnp.int32),
            pltpu.SMEM((2 * NBKT,), jnp.int32),
            pltpu.SemaphoreType.DMA,
            pltpu.SemaphoreType.DMA,
            pltpu.SemaphoreType.DMA,
        ],
        compiler_params=pltpu.CompilerParams(needs_layout_passes=False),
    )(idx3, ent_t, rel_t, tail_e, tail_r)

    return pl.kernel(
        _phase2, mesh=mesh,
        out_type=jax.ShapeDtypeStruct((BATCH,), jnp.float32),
        scratch_types=[
            pltpu.VMEM((C2, 2 * DIM), jnp.float32),
            pltpu.VMEM((C2, 2 * DIM), jnp.float32),
            pltpu.VMEM((C2, 2 * DIM), jnp.float32),
            pltpu.VMEM((2, 128, L), jnp.int32),
            pltpu.VMEM((3 * B_PER_W,), jnp.int32),
            pltpu.VMEM((C2,), jnp.int32),
            pltpu.VMEM((C2,), jnp.int32),
            pltpu.VMEM((C2,), jnp.int32),
            pltpu.VMEM((B_PER_W,), jnp.float32),
            pltpu.SemaphoreType.DMA,
            pltpu.SemaphoreType.DMA,
        ],
        compiler_params=pltpu.CompilerParams(needs_layout_passes=False),
    )(rows, slots)
